# baseline (speedup 1.0000x reference)
import functools

import jax
import jax.numpy as jnp
from jax import lax
from jax.experimental import pallas as pl
from jax.experimental.pallas import tpu as pltpu
from jax.experimental.pallas import tpu_sc as plsc

N_IDS = 16384
NUM_CORES = 2
NUM_SUBCORES = 16
NUM_WORKERS = NUM_CORES * NUM_SUBCORES
ROWS_PER_WORKER = N_IDS // NUM_WORKERS

PART_WIDTHS = (16, 56, 56)
OUT_D = 128

CONCAT_ROWS = 2048


def _sc_gather(ids_by_axis, t0, t12):
    b = ROWS_PER_WORKER
    mesh = plsc.VectorSubcoreMesh(core_axis_name="c", subcore_axis_name="s")

    scratch = [pltpu.VMEM((b,), jnp.int32) for _ in range(3)]
    scratch += [
        pltpu.VMEM((b, 16), jnp.float32),
        pltpu.VMEM((b, 112), jnp.float32),
        pltpu.VMEM((b, 112), jnp.float32),
    ]
    scratch += [pltpu.SemaphoreType.DMA]

    @functools.partial(
        pl.kernel,
        out_type=tuple(
            jax.ShapeDtypeStruct((N_IDS, w), jnp.float32)
            for w in PART_WIDTHS
        ),
        mesh=mesh,
        scratch_types=scratch,
        compiler_params=pltpu.CompilerParams(use_tc_tiling_on_sc=False),
    )
    def body(ids0_hbm, ids1_hbm, ids2_hbm, tab0, tab12,
             o0, o1, o2, i0, i1, i2, b0, b1, b2, sem):
        wid = lax.axis_index("s") * NUM_CORES + lax.axis_index("c")
        base = wid * b
        idxs = (i0, i1, i2)
        for ax, ids_hbm in enumerate((ids0_hbm, ids1_hbm, ids2_hbm)):
            pltpu.sync_copy(ids_hbm.at[pl.ds(base, b)], idxs[ax])
        copies = [
            pltpu.async_copy(tab0.at[i0], b0, sem),
            pltpu.async_copy(tab12.at[i1], b1, sem),
            pltpu.async_copy(tab12.at[i2], b2, sem),
        ]
        for cp in copies:
            cp.wait()
        pltpu.sync_copy(b0, o0.at[pl.ds(base, b), :])
        pltpu.sync_copy(b1.at[:, pl.ds(0, 56)], o1.at[pl.ds(base, b), :])
        pltpu.sync_copy(b2.at[:, pl.ds(56, 56)], o2.at[pl.ds(base, b), :])

    return body(*ids_by_axis, t0, t12)


def _tc_shuffle(parts):
    def body(g0, g1, g2, out_ref):
        out_ref[...] = jnp.concatenate(
            [
                g0[:, 0:8],
                g1[:, 0:28],
                g2[:, 0:28],
                g0[:, 8:16],
                g1[:, 28:56],
                g2[:, 28:56],
            ],
            axis=-1,
        )

    grid = (N_IDS // CONCAT_ROWS,)
    in_specs = [
        pl.BlockSpec((CONCAT_ROWS, w), lambda i: (i, 0))
        for w in PART_WIDTHS
    ]
    return pl.pallas_call(
        body,
        out_shape=jax.ShapeDtypeStruct((N_IDS, OUT_D), jnp.float32),
        grid=grid,
        in_specs=in_specs,
        out_specs=pl.BlockSpec((CONCAT_ROWS, OUT_D), lambda i: (i, 0)),
    )(*parts)


def kernel(ids, cos_0, cos_1, cos_2, sin_0, sin_1, sin_2):
    ids_by_axis = (ids[:, 0], ids[:, 1], ids[:, 2])
    t0 = jnp.concatenate([cos_0, sin_0], axis=1)
    t12 = jnp.concatenate([cos_1, sin_1, cos_2, sin_2], axis=1)
    t12 = jnp.concatenate([t12, jnp.zeros((73728 - 8192, 112), jnp.float32)], axis=0)
    t0 = jnp.concatenate([t0, jnp.zeros((73728 - 8192, 16), jnp.float32)], axis=0)
    parts = _sc_gather(ids_by_axis, t0, t12)
    return _tc_shuffle(parts)


# speedup vs baseline: 2.4456x; 2.4456x over previous
import functools

import jax
import jax.numpy as jnp
from jax import lax
from jax.experimental import pallas as pl
from jax.experimental.pallas import tpu as pltpu
from jax.experimental.pallas import tpu_sc as plsc

N_IDS = 16384
NUM_CORES = 2
NUM_SUBCORES = 16
NUM_WORKERS = NUM_CORES * NUM_SUBCORES
ROWS_PER_WORKER = N_IDS // NUM_WORKERS

GATHER_WIDTHS = (16, 64, 64)
OUT_D = 128

CONCAT_ROWS = 2048


def _sc_gather(ids_by_axis, tables):
    b = ROWS_PER_WORKER
    mesh = plsc.VectorSubcoreMesh(core_axis_name="c", subcore_axis_name="s")

    scratch = [pltpu.VMEM((b,), jnp.int32) for _ in range(3)]
    scratch += [pltpu.VMEM((b, w), jnp.bfloat16) for w in GATHER_WIDTHS]
    scratch += [pltpu.SemaphoreType.DMA]

    @functools.partial(
        pl.kernel,
        out_type=tuple(
            jax.ShapeDtypeStruct((N_IDS, w), jnp.bfloat16)
            for w in GATHER_WIDTHS
        ),
        mesh=mesh,
        scratch_types=scratch,
        compiler_params=pltpu.CompilerParams(use_tc_tiling_on_sc=False),
    )
    def body(ids0_hbm, ids1_hbm, ids2_hbm, w0, w1, w2,
             o0, o1, o2, i0, i1, i2, b0, b1, b2, sem):
        wid = lax.axis_index("s") * NUM_CORES + lax.axis_index("c")
        base = wid * b
        idxs = (i0, i1, i2)
        for ax, ids_hbm in enumerate((ids0_hbm, ids1_hbm, ids2_hbm)):
            pltpu.sync_copy(ids_hbm.at[pl.ds(base, b)], idxs[ax])
        copies = []
        for t, buf, idx in zip((w0, w1, w2), (b0, b1, b2), idxs):
            copies.append(pltpu.async_copy(t.at[idx], buf, sem))
        for cp in copies:
            cp.wait()
        for buf, out in zip((b0, b1, b2), (o0, o1, o2)):
            pltpu.sync_copy(buf, out.at[pl.ds(base, b), :])

    return body(*ids_by_axis, *tables)


def _tc_shuffle(parts):
    def body(g0, g1, g2, out_ref):
        g0, g1, g2 = (x[...].astype(jnp.float32) for x in (g0, g1, g2))
        out_ref[...] = jnp.concatenate(
            [
                g0[:, 0:8], g1[:, 0:28], g2[:, 0:28],
                g0[:, 8:16], g1[:, 28:56], g2[:, 28:56],
            ],
            axis=-1,
        )

    grid = (N_IDS // CONCAT_ROWS,)
    in_specs = [
        pl.BlockSpec((CONCAT_ROWS, w), lambda i: (i, 0))
        for w in GATHER_WIDTHS
    ]
    return pl.pallas_call(
        body,
        out_shape=jax.ShapeDtypeStruct((N_IDS, OUT_D), jnp.float32),
        grid=grid,
        in_specs=in_specs,
        out_specs=pl.BlockSpec((CONCAT_ROWS, OUT_D), lambda i: (i, 0)),
    )(*parts)


def kernel(ids, cos_0, cos_1, cos_2, sin_0, sin_1, sin_2):
    ids_by_axis = (ids[:, 0], ids[:, 1], ids[:, 2])
    pad = jnp.zeros((cos_1.shape[0], 8), jnp.float32)
    tables = tuple(
        t.astype(jnp.bfloat16) for t in (
            jnp.concatenate([cos_0, sin_0], axis=1),
            jnp.concatenate([cos_1, sin_1, pad], axis=1),
            jnp.concatenate([cos_2, sin_2, pad], axis=1),
        )
    )
    parts = _sc_gather(ids_by_axis, tables)
    return _tc_shuffle(parts)


# speedup vs baseline: 2.5369x; 1.0373x over previous
import functools

import jax
import jax.numpy as jnp
from jax import lax
from jax.experimental import pallas as pl
from jax.experimental.pallas import tpu as pltpu
from jax.experimental.pallas import tpu_sc as plsc

N_IDS = 16384
NUM_CORES = 2
NUM_SUBCORES = 16
NUM_WORKERS = NUM_CORES * NUM_SUBCORES
ROWS_PER_WORKER = N_IDS // NUM_WORKERS

GATHER_WIDTHS = (128, 128, 128)
CHUNK = ROWS_PER_WORKER // 2
OUT_D = 128

CONCAT_ROWS = 2048


def _sc_gather(ids_by_axis, table):
    b = ROWS_PER_WORKER
    mesh = plsc.VectorSubcoreMesh(core_axis_name="c", subcore_axis_name="s")

    scratch = [pltpu.VMEM((b,), jnp.int32) for _ in range(3)]
    scratch += [pltpu.VMEM((CHUNK, 128), jnp.float32) for _ in range(3)]
    scratch += [pltpu.SemaphoreType.DMA]

    @functools.partial(
        pl.kernel,
        out_type=tuple(
            jax.ShapeDtypeStruct((N_IDS, w), jnp.float32)
            for w in GATHER_WIDTHS
        ),
        mesh=mesh,
        scratch_types=scratch,
        compiler_params=pltpu.CompilerParams(use_tc_tiling_on_sc=False),
    )
    def body(ids0_hbm, ids1_hbm, ids2_hbm, tab,
             o0, o1, o2, i0, i1, i2, b0, b1, b2, sem):
        wid = lax.axis_index("s") * NUM_CORES + lax.axis_index("c")
        base = wid * b
        idxs = (i0, i1, i2)
        for ax, ids_hbm in enumerate((ids0_hbm, ids1_hbm, ids2_hbm)):
            pltpu.sync_copy(ids_hbm.at[pl.ds(base, b)], idxs[ax])
        for half in range(2):
            lo = half * CHUNK
            copies = []
            for buf, idx in zip((b0, b1, b2), idxs):
                copies.append(pltpu.async_copy(
                    tab.at[idx.at[pl.ds(lo, CHUNK)]], buf, sem))
            for cp in copies:
                cp.wait()
            for buf, out in zip((b0, b1, b2), (o0, o1, o2)):
                pltpu.sync_copy(buf, out.at[pl.ds(base + lo, CHUNK), :])

    return body(*ids_by_axis, table)


def _tc_shuffle(parts):
    def body(g0, g1, g2, out_ref):
        out_ref[...] = jnp.concatenate(
            [
                g0[:, 0:8],
                g1[:, 16:44],
                g2[:, 72:100],
                g0[:, 8:16],
                g1[:, 44:72],
                g2[:, 100:128],
            ],
            axis=-1,
        )

    grid = (N_IDS // CONCAT_ROWS,)
    in_specs = [
        pl.BlockSpec((CONCAT_ROWS, w), lambda i: (i, 0))
        for w in GATHER_WIDTHS
    ]
    return pl.pallas_call(
        body,
        out_shape=jax.ShapeDtypeStruct((N_IDS, OUT_D), jnp.float32),
        grid=grid,
        in_specs=in_specs,
        out_specs=pl.BlockSpec((CONCAT_ROWS, OUT_D), lambda i: (i, 0)),
    )(*parts)


def kernel(ids, cos_0, cos_1, cos_2, sin_0, sin_1, sin_2):
    ids_by_axis = (ids[:, 0], ids[:, 1], ids[:, 2])
    table = jnp.concatenate([cos_0, sin_0, cos_1, sin_1, cos_2, sin_2],
                            axis=1)
    parts = _sc_gather(ids_by_axis, table)
    return _tc_shuffle(parts)
